# baseline (speedup 1.0000x reference)
import dataclasses
import functools

import jax
import jax.numpy as jnp
from jax import lax
from jax.experimental import pallas as pl
from jax.experimental.pallas import tpu as pltpu
from jax.experimental.pallas import tpu_sc as plsc

R = 4096
C = 1000
K = 200
L = 16
NW = 32
ROWS_PER_W = R // NW
BLK = 16
PAIRS = BLK // 2
CHUNKS_PER_PAIR = (2 * K) // L


def _sc_gather(xf, idxf):
    mesh = plsc.VectorSubcoreMesh(core_axis_name="c", subcore_axis_name="s")
    cp = pltpu.CompilerParams()
    if "needs_layout_passes" in pltpu.CompilerParams.__dataclass_fields__:
        cp = dataclasses.replace(cp, needs_layout_passes=False)

    @functools.partial(
        pl.kernel,
        out_type=jax.ShapeDtypeStruct((R * K,), jnp.float32),
        mesh=mesh,
        compiler_params=cp,
        scratch_types=[
            pltpu.VMEM((BLK * C,), jnp.float32),
            pltpu.VMEM((BLK * K,), jnp.int32),
            pltpu.VMEM((BLK * K,), jnp.float32),
        ],
    )
    def k(x_hbm, i_hbm, o_hbm, xv, iv, ov):
        wid = lax.axis_index("s") * 2 + lax.axis_index("c")
        row0 = wid * ROWS_PER_W
        ii = lax.iota(jnp.int32, L)

        @pl.loop(0, ROWS_PER_W // BLK)
        def _(b):
            base_row = row0 + b * BLK
            pltpu.sync_copy(x_hbm.at[pl.ds(base_row * C, BLK * C)], xv)
            pltpu.sync_copy(i_hbm.at[pl.ds(base_row * K, BLK * K)], iv)

            @pl.loop(0, PAIRS)
            def _(p):
                rbase0 = p * (2 * C)
                rbase1 = rbase0 + C
                for kk in range(CHUNKS_PER_PAIR):
                    off = p * (2 * K) + kk * L
                    idx_chunk = iv[pl.ds(off, L)]
                    if kk < 12:
                        flat = idx_chunk + rbase0
                    elif kk == 12:
                        flat = idx_chunk + jnp.where(ii >= 8, rbase1, rbase0)
                    else:
                        flat = idx_chunk + rbase1
                    ov[pl.ds(off, L)] = plsc.load_gather(xv, [flat])

            pltpu.sync_copy(ov, o_hbm.at[pl.ds(base_row * K, BLK * K)])

    return k(xf, idxf)


def kernel(x, indices):
    xf = x.reshape(-1)
    idxf = indices.astype(jnp.int32).reshape(-1)
    out = _sc_gather(xf, idxf)
    return out.reshape(R, K)


# speedup vs baseline: 1.4455x; 1.4455x over previous
import dataclasses
import functools

import jax
import jax.numpy as jnp
from jax import lax
from jax.experimental import pallas as pl
from jax.experimental.pallas import tpu as pltpu
from jax.experimental.pallas import tpu_sc as plsc

R = 4096
C = 1000
K = 200
L = 16
NW = 32
ROWS_PER_W = R // NW
BLK = 16
CHUNK_OFFS = tuple(range(0, K - L + 1, L)) + (K - L,)


def _sc_gather(x, idx):
    mesh = plsc.VectorSubcoreMesh(core_axis_name="c", subcore_axis_name="s")
    cp = pltpu.CompilerParams()
    if "needs_layout_passes" in pltpu.CompilerParams.__dataclass_fields__:
        cp = dataclasses.replace(cp, needs_layout_passes=False)

    @functools.partial(
        pl.kernel,
        out_type=jax.ShapeDtypeStruct((R, K), jnp.float32),
        mesh=mesh,
        compiler_params=cp,
        scratch_types=[
            pltpu.VMEM((BLK, C), jnp.float32),
            pltpu.VMEM((BLK, K), jnp.int32),
            pltpu.VMEM((BLK, K), jnp.float32),
        ],
    )
    def k(x_hbm, i_hbm, o_hbm, xv, iv, ov):
        wid = lax.axis_index("s") * 2 + lax.axis_index("c")
        row0 = wid * ROWS_PER_W

        @pl.loop(0, ROWS_PER_W // BLK)
        def _(b):
            base_row = row0 + b * BLK
            pltpu.sync_copy(x_hbm.at[pl.ds(base_row, BLK)], xv)
            pltpu.sync_copy(i_hbm.at[pl.ds(base_row, BLK)], iv)

            @pl.loop(0, BLK)
            def _(r):
                rvec = jnp.full((L,), 0, jnp.int32) + r
                for off in CHUNK_OFFS:
                    cols = iv[r, pl.ds(off, L)]
                    ov[r, pl.ds(off, L)] = plsc.load_gather(xv, [rvec, cols])

            pltpu.sync_copy(ov, o_hbm.at[pl.ds(base_row, BLK)])

    return k(x, idx)


def kernel(x, indices):
    return _sc_gather(x, indices.astype(jnp.int32))


# speedup vs baseline: 1.7763x; 1.2288x over previous
import dataclasses
import functools

import jax
import jax.numpy as jnp
from jax import lax
from jax.experimental import pallas as pl
from jax.experimental.pallas import tpu as pltpu
from jax.experimental.pallas import tpu_sc as plsc

R = 4096
C = 1000
K = 200
L = 16
NW = 32
ROWS_PER_W = R // NW
BLK = 16
NB = ROWS_PER_W // BLK
CHUNK_OFFS = tuple(range(0, K - L + 1, L)) + (K - L,)


def _sc_gather(x, idx):
    mesh = plsc.VectorSubcoreMesh(core_axis_name="c", subcore_axis_name="s")
    cp = pltpu.CompilerParams()
    if "needs_layout_passes" in pltpu.CompilerParams.__dataclass_fields__:
        cp = dataclasses.replace(cp, needs_layout_passes=False)

    @functools.partial(
        pl.kernel,
        out_type=jax.ShapeDtypeStruct((R, K), jnp.float32),
        mesh=mesh,
        compiler_params=cp,
        scratch_types=[
            pltpu.VMEM((BLK, C), jnp.float32),
            pltpu.VMEM((BLK, C), jnp.float32),
            pltpu.VMEM((ROWS_PER_W, K), jnp.int32),
            pltpu.VMEM((ROWS_PER_W, K), jnp.float32),
            pltpu.SemaphoreType.DMA,
            pltpu.SemaphoreType.DMA,
        ],
    )
    def k(x_hbm, i_hbm, o_hbm, xv0, xv1, iv, ov, sx0, sx1):
        wid = lax.axis_index("s") * 2 + lax.axis_index("c")
        row0 = wid * ROWS_PER_W

        def x_copy(b, buf, sem):
            return pltpu.make_async_copy(
                x_hbm.at[pl.ds(row0 + b * BLK, BLK)], buf, sem)

        def gather_block(b, buf):
            @pl.loop(0, BLK)
            def _(r):
                rvec = jnp.full((L,), 0, jnp.int32) + r
                orow = b * BLK + r
                for off in CHUNK_OFFS:
                    cols = iv[orow, pl.ds(off, L)]
                    ov[orow, pl.ds(off, L)] = plsc.load_gather(buf, [rvec, cols])

        x_copy(0, xv0, sx0).start()
        pltpu.sync_copy(i_hbm.at[pl.ds(row0, ROWS_PER_W)], iv)

        @pl.loop(0, NB // 2)
        def _(g):
            b0 = 2 * g
            x_copy(b0 + 1, xv1, sx1).start()
            x_copy(b0, xv0, sx0).wait()
            gather_block(b0, xv0)

            @pl.when(b0 + 2 < NB)
            def _():
                x_copy(b0 + 2, xv0, sx0).start()

            x_copy(b0 + 1, xv1, sx1).wait()
            gather_block(b0 + 1, xv1)

        pltpu.sync_copy(ov, o_hbm.at[pl.ds(row0, ROWS_PER_W)])

    return k(x, idx)


def kernel(x, indices):
    return _sc_gather(x, indices.astype(jnp.int32))
